# baseline (speedup 1.0000x reference)
import functools
import math

import jax
import jax.numpy as jnp
from jax import lax
from jax.experimental import pallas as pl
from jax.experimental.pallas import tpu as pltpu
from jax.experimental.pallas import tpu_sc as plsc

D_MODEL = 64
SCALE = math.sqrt(D_MODEL)
NC = 2
NS = 16
NW = NC * NS
CH = 128
NBUF = 5
L = 16
RU = 8


@functools.lru_cache(maxsize=None)
def _make(H, BSZ):
    cols_per_w = BSZ // NW
    sub_per_h = cols_per_w // CH
    nchunk = H * sub_per_h
    mesh = plsc.VectorSubcoreMesh(core_axis_name="c", subcore_axis_name="s")

    @functools.partial(
        pl.kernel,
        mesh=mesh,
        compiler_params=pltpu.CompilerParams(use_tc_tiling_on_sc=True),
        out_type=jax.ShapeDtypeStruct((H, BSZ, D_MODEL), jnp.float32),
        scratch_types=[
            pltpu.VMEM((H, cols_per_w), jnp.int32),
            pltpu.VMEM((NBUF, CH, D_MODEL), jnp.float32),
        ]
        + [pltpu.SemaphoreType.DMA] * (2 * NBUF),
    )
    def emb(idx_hbm, table_hbm, out_hbm, idx_v, rows_v, *sems):
        gsem = sems[:NBUF]
        osem = sems[NBUF:]
        wid = lax.axis_index("s") * NC + lax.axis_index("c")
        col0 = wid * cols_per_w

        pltpu.sync_copy(idx_hbm.at[:, pl.ds(col0, cols_per_w)], idx_v)

        def start_gather(c, b):
            h = c // sub_per_h
            base = (c % sub_per_h) * CH

            def issue(k, carry):
                v = idx_v[h, pl.ds(base + k * L, L)]
                for lane in range(L):
                    pltpu.async_copy(
                        table_hbm.at[0, v[lane]],
                        rows_v.at[b, k * L + lane],
                        gsem[b],
                    )
                return carry

            lax.fori_loop(0, CH // L, issue, 0)

        def wait_gather(b):
            pltpu.make_async_copy(
                table_hbm.at[0, pl.ds(0, CH)], rows_v.at[b], gsem[b]
            ).wait()

        def out_slice(c):
            h = c // sub_per_h
            base = col0 + (c % sub_per_h) * CH
            return out_hbm.at[h, pl.ds(base, CH)]

        def scale_chunk(b):
            def scale_rows(r, carry):
                for rr in range(RU):
                    row = r * RU + rr
                    for j in range(D_MODEL // L):
                        sl = (row, pl.ds(j * L, L))
                        rows_v[b, *sl] = rows_v[b, *sl] * SCALE
                return carry

            lax.fori_loop(0, CH // RU, scale_rows, 0)

        for b in range(NBUF):
            start_gather(b, b)

        def chunk_body(c, b, issue_next):
            wait_gather(b)
            scale_chunk(b)
            pltpu.async_copy(rows_v.at[b], out_slice(c), osem[b])
            if issue_next:
                pltpu.make_async_copy(rows_v.at[b], out_slice(c), osem[b]).wait()
                start_gather(c + NBUF, b)
            return 0

        def steady(t, carry):
            for b in range(NBUF):
                chunk_body(t * NBUF + b, b, True)
            return carry

        lax.fori_loop(0, nchunk // NBUF - 1, steady, 0)
        for b in range(NBUF):
            chunk_body(nchunk - NBUF + b, b, False)
        for b in range(NBUF):
            c = nchunk - NBUF + b
            pltpu.make_async_copy(rows_v.at[b], out_slice(c), osem[b]).wait()

    return emb


def kernel(indices, table):
    bsz, hist = indices.shape
    nodes, d = table.shape
    table3 = table.reshape(1, nodes, d)
    out_t = _make(hist, bsz)(indices.T.astype(jnp.int32), table3)
    return out_t.transpose(1, 0, 2)


# speedup vs baseline: 1.0003x; 1.0003x over previous
import functools
import math

import jax
import jax.numpy as jnp
from jax import lax
from jax.experimental import pallas as pl
from jax.experimental.pallas import tpu as pltpu
from jax.experimental.pallas import tpu_sc as plsc

D_MODEL = 64
SCALE = math.sqrt(D_MODEL)
NC = 2
NS = 16
NW = NC * NS
CH = 256
NBUF = 2
L = 16
RU = 8


@functools.lru_cache(maxsize=None)
def _make(H, BSZ):
    cols_per_w = BSZ // NW
    sub_per_h = cols_per_w // CH
    nchunk = H * sub_per_h
    mesh = plsc.VectorSubcoreMesh(core_axis_name="c", subcore_axis_name="s")

    @functools.partial(
        pl.kernel,
        mesh=mesh,
        compiler_params=pltpu.CompilerParams(use_tc_tiling_on_sc=True),
        out_type=jax.ShapeDtypeStruct((H, BSZ, D_MODEL), jnp.float32),
        scratch_types=[
            pltpu.VMEM((H, cols_per_w), jnp.int32),
            pltpu.VMEM((NBUF, CH, D_MODEL), jnp.float32),
        ]
        + [pltpu.SemaphoreType.DMA] * (2 * NBUF),
    )
    def emb(idx_hbm, table_hbm, out_hbm, idx_v, rows_v, *sems):
        gsem = sems[:NBUF]
        osem = sems[NBUF:]
        wid = lax.axis_index("s") * NC + lax.axis_index("c")
        col0 = wid * cols_per_w

        pltpu.sync_copy(idx_hbm.at[:, pl.ds(col0, cols_per_w)], idx_v)

        def start_gather(c, b):
            h = c // sub_per_h
            base = (c % sub_per_h) * CH

            def issue(k, carry):
                v = idx_v[h, pl.ds(base + k * L, L)]
                for lane in range(L):
                    pltpu.async_copy(
                        table_hbm.at[0, v[lane]],
                        rows_v.at[b, k * L + lane],
                        gsem[b],
                    )
                return carry

            lax.fori_loop(0, CH // L, issue, 0)

        def wait_gather(b):
            pltpu.make_async_copy(
                table_hbm.at[0, pl.ds(0, CH)], rows_v.at[b], gsem[b]
            ).wait()

        def out_slice(c):
            h = c // sub_per_h
            base = col0 + (c % sub_per_h) * CH
            return out_hbm.at[h, pl.ds(base, CH)]

        def scale_chunk(b):
            def scale_rows(r, carry):
                for rr in range(RU):
                    row = r * RU + rr
                    for j in range(D_MODEL // L):
                        sl = (row, pl.ds(j * L, L))
                        rows_v[b, *sl] = rows_v[b, *sl] * SCALE
                return carry

            lax.fori_loop(0, CH // RU, scale_rows, 0)

        for b in range(NBUF):
            start_gather(b, b)

        def chunk_body(c, b, issue_next):
            wait_gather(b)
            scale_chunk(b)
            pltpu.async_copy(rows_v.at[b], out_slice(c), osem[b])
            if issue_next:
                pltpu.make_async_copy(rows_v.at[b], out_slice(c), osem[b]).wait()
                start_gather(c + NBUF, b)
            return 0

        def steady(t, carry):
            for b in range(NBUF):
                chunk_body(t * NBUF + b, b, True)
            return carry

        lax.fori_loop(0, nchunk // NBUF - 1, steady, 0)
        for b in range(NBUF):
            chunk_body(nchunk - NBUF + b, b, False)
        for b in range(NBUF):
            c = nchunk - NBUF + b
            pltpu.make_async_copy(rows_v.at[b], out_slice(c), osem[b]).wait()

    return emb


def kernel(indices, table):
    bsz, hist = indices.shape
    nodes, d = table.shape
    table3 = table.reshape(1, nodes, d)
    out_t = _make(hist, bsz)(indices.T.astype(jnp.int32), table3)
    return out_t.transpose(1, 0, 2)
